# baseline (speedup 1.0000x reference)
import functools

import jax
import jax.numpy as jnp
from jax import lax
from jax.experimental import pallas as pl
from jax.experimental.pallas import tpu as pltpu
from jax.experimental.pallas import tpu_sc as plsc

B = 16384
K = 16
NC = 2
NS = 16
NW = NC * NS
BPW = B // NW
CH = 128
NCH = BPW // CH
G = BPW // 16

_mesh = plsc.VectorSubcoreMesh(core_axis_name="c", subcore_axis_name="s")


@functools.partial(
    pl.kernel,
    out_type=jax.ShapeDtypeStruct((B,), jnp.float32),
    mesh=_mesh,
    scratch_types=[
        pltpu.VMEM((NCH, CH), jnp.int32),
        pltpu.VMEM((NCH, CH), jnp.int32),
        pltpu.VMEM((BPW, K), jnp.float32),
        pltpu.VMEM((BPW, K), jnp.float32),
        pltpu.VMEM((BPW,), jnp.float32),
        pltpu.SemaphoreType.DMA,
    ],
    compiler_params=pltpu.CompilerParams(
        needs_layout_passes=False, use_tc_tiling_on_sc=False),
)
def _bpr_sc(uid2d, iid2d, umat, imat, out, uidx, iidx, urows, vrows, outv, sem):
    wid = lax.axis_index("s") * NC + lax.axis_index("c")
    pltpu.sync_copy(uid2d.at[pl.ds(wid * NCH, NCH)], uidx)
    pltpu.sync_copy(iid2d.at[pl.ds(wid * NCH, NCH)], iidx)
    copies = []
    for j in range(NCH):
        copies.append(pltpu.async_copy(
            umat.at[uidx.at[j]], urows.at[pl.ds(j * CH, CH)], sem))
        copies.append(pltpu.async_copy(
            imat.at[iidx.at[j]], vrows.at[pl.ds(j * CH, CH)], sem))
    for c in copies:
        c.wait()

    lane = lax.iota(jnp.int32, 16)

    def group(g, _):
        rows = g * 16 + lane
        acc = jnp.zeros((16,), jnp.float32)
        for k in range(K):
            col = jnp.full((16,), k, jnp.int32)
            uc = plsc.load_gather(urows, [rows, col])
            vc = plsc.load_gather(vrows, [rows, col])
            acc = acc + uc * vc
        outv[pl.ds(g * 16, 16)] = acc
        return 0

    lax.fori_loop(0, G, group, 0)
    pltpu.sync_copy(outv, out.at[pl.ds(wid * BPW, BPW)])


def kernel(uid, iid, user_mat, item_mat):
    uid2d = uid.astype(jnp.int32).reshape(NW * NCH, CH)
    iid2d = iid.astype(jnp.int32).reshape(NW * NCH, CH)
    return _bpr_sc(uid2d, iid2d, user_mat, item_mat)
